# baseline (speedup 1.0000x reference)
import functools

import jax
import jax.numpy as jnp
from jax import lax
from jax.experimental import pallas as pl
from jax.experimental.pallas import tpu as pltpu
from jax.experimental.pallas import tpu_sc as plsc

NC = 2
NS = 16
NW = NC * NS
CH = 100
RING = 4
NACC = 4


def _make_sc_pool(B, L, D):
    rows_per_w = B // NW
    nchunk = L // CH
    idx_rows_per_w = rows_per_w * nchunk

    mesh = plsc.VectorSubcoreMesh(core_axis_name="c", subcore_axis_name="s")

    @functools.partial(
        pl.kernel,
        mesh=mesh,
        out_type=jax.ShapeDtypeStruct((B, D), jnp.float32),
        compiler_params=pltpu.CompilerParams(use_tc_tiling_on_sc=False),
        scratch_types=[
            pltpu.VMEM((idx_rows_per_w, CH), jnp.int32),
            pltpu.VMEM((rows_per_w, D), jnp.float32),
        ]
        + [pltpu.VMEM((L, D), jnp.float32) for _ in range(RING)]
        + [pltpu.SemaphoreType.DMA for _ in range(RING)],
    )
    def sc_pool(idx_hbm, table_hbm, out_hbm, idx_v, pooled_v, *bufs_and_sems):
        bufs = bufs_and_sems[:RING]
        sems = bufs_and_sems[RING:]
        wid = lax.axis_index("s") * NC + lax.axis_index("c")
        pltpu.sync_copy(idx_hbm.at[pl.ds(wid * idx_rows_per_w,
                                         idx_rows_per_w)], idx_v)

        def start_row(row, slot):
            for c in range(nchunk):
                pltpu.make_async_copy(
                    table_hbm.at[idx_v.at[nchunk * row + c]],
                    bufs[slot].at[pl.ds(c * CH, CH)],
                    sems[slot]).start()

        def finish_row(row, slot):
            for c in range(nchunk):
                pltpu.make_async_copy(
                    table_hbm.at[idx_v.at[nchunk * row + c]],
                    bufs[slot].at[pl.ds(c * CH, CH)],
                    sems[slot]).wait()
            buf = bufs[slot]
            accs = [buf[a] for a in range(NACC)]
            for j in range(NACC, L):
                accs[j % NACC] = accs[j % NACC] + buf[j]
            total = (accs[0] + accs[1]) + (accs[2] + accs[3])
            pooled_v[row] = total

        for r in range(RING):
            start_row(r, r)

        def group_body(g, carry):
            for r in range(RING):
                row = g * RING + r
                finish_row(row, r)
                start_row(row + RING, r)
            return carry

        lax.fori_loop(0, rows_per_w // RING - 1, group_body, 0)

        for r in range(RING):
            finish_row(rows_per_w - RING + r, r)

        pltpu.sync_copy(pooled_v, out_hbm.at[pl.ds(wid * rows_per_w,
                                                   rows_per_w)])

    return sc_pool


def _tr_body(t_ref, out_ref):
    out_ref[...] = t_ref[...].T


def _transpose_table(tT, D):
    V = tT.shape[1]
    C = 2048
    grid = (V + C - 1) // C
    return pl.pallas_call(
        _tr_body,
        grid=(grid,),
        in_specs=[pl.BlockSpec((D, C), lambda g: (0, g))],
        out_specs=pl.BlockSpec((C, D), lambda g: (g, 0)),
        out_shape=jax.ShapeDtypeStruct((V, D), jnp.float32),
    )(tT)


def _dense_body(pooled_ref, w1_ref, b1_ref, w2_ref, b2_ref, out_ref):
    p = pooled_ref[...]
    h = jnp.maximum(
        jnp.dot(p, w1_ref[...], preferred_element_type=jnp.float32)
        + b1_ref[...], 0.0)
    z = jnp.dot(h, w2_ref[...], preferred_element_type=jnp.float32) + b2_ref[...]
    out_ref[...] = 1.0 / (1.0 + jnp.exp(-z))


@jax.jit
def kernel(indices, table, W1, b1, W2, b2):
    B, L = indices.shape
    D = table.shape[1]
    n_class = W2.shape[1]

    idx2 = indices.astype(jnp.int32).reshape(B * L // CH, CH)
    table_rm = _transpose_table(table.T, D)
    sums = _make_sc_pool(B, L, D)(idx2, table_rm)

    out = pl.pallas_call(
        _dense_body,
        out_shape=jax.ShapeDtypeStruct((B, n_class), jnp.float32),
    )(sums, W1 * (1.0 / L), b1.reshape(1, D), W2, b2.reshape(1, n_class))
    return out


# speedup vs baseline: 6.4703x; 6.4703x over previous
import functools

import jax
import jax.numpy as jnp
from jax import lax
from jax.experimental import pallas as pl
from jax.experimental.pallas import tpu as pltpu
from jax.experimental.pallas import tpu_sc as plsc

NC = 2
NS = 16
NW = NC * NS
CHUNKS = ((0, 104), (104, 96))
RING = 4
NACC = 4


def _make_sc_pool(B, L, D):
    rows_per_w = B // NW
    flat_per_w = rows_per_w * L

    mesh = plsc.VectorSubcoreMesh(core_axis_name="c", subcore_axis_name="s")

    @functools.partial(
        pl.kernel,
        mesh=mesh,
        out_type=jax.ShapeDtypeStruct((B, D), jnp.float32),
        compiler_params=pltpu.CompilerParams(use_tc_tiling_on_sc=False),
        scratch_types=[
            pltpu.VMEM((flat_per_w,), jnp.int32),
            pltpu.VMEM((rows_per_w, D), jnp.float32),
        ]
        + [pltpu.VMEM((L, D), jnp.float32) for _ in range(RING)]
        + [pltpu.SemaphoreType.DMA for _ in range(RING)],
    )
    def sc_pool(idx_hbm, table_hbm, out_hbm, idx_v, pooled_v, *bufs_and_sems):
        bufs = bufs_and_sems[:RING]
        sems = bufs_and_sems[RING:]
        wid = lax.axis_index("s") * NC + lax.axis_index("c")
        pltpu.sync_copy(idx_hbm.at[pl.ds(wid * flat_per_w, flat_per_w)], idx_v)

        def start_row(row, slot):
            for off, n in CHUNKS:
                pltpu.make_async_copy(
                    table_hbm.at[idx_v.at[pl.ds(row * L + off, n)]],
                    bufs[slot].at[pl.ds(off, n)],
                    sems[slot]).start()

        def finish_row(row, slot):
            for off, n in CHUNKS:
                pltpu.make_async_copy(
                    table_hbm.at[idx_v.at[pl.ds(row * L + off, n)]],
                    bufs[slot].at[pl.ds(off, n)],
                    sems[slot]).wait()
            buf = bufs[slot]
            accs = [buf[a] for a in range(NACC)]
            for j in range(NACC, L):
                accs[j % NACC] = accs[j % NACC] + buf[j]
            total = (accs[0] + accs[1]) + (accs[2] + accs[3])
            pooled_v[row] = total

        for r in range(RING):
            start_row(r, r)

        def group_body(g, carry):
            for r in range(RING):
                row = g * RING + r
                finish_row(row, r)
                start_row(row + RING, r)
            return carry

        lax.fori_loop(0, rows_per_w // RING - 1, group_body, 0)

        for r in range(RING):
            finish_row(rows_per_w - RING + r, r)

        pltpu.sync_copy(pooled_v, out_hbm.at[pl.ds(wid * rows_per_w,
                                                   rows_per_w)])

    return sc_pool


def _tr_body(t_ref, out_ref):
    x = t_ref[...]
    tiles = []
    for g8 in range(x.shape[1] // 1024):
        base = g8 * 1024
        p = jnp.concatenate(
            [x[:, base + 128 * c: base + 128 * (c + 1)] for c in range(8)],
            axis=0)
        tiles.append(p.T)
    out_ref[...] = jnp.concatenate(tiles, axis=0)


def _transpose_table(tT, D):
    V = tT.shape[1]
    C = 65536
    grid = (V + C - 1) // C
    out = pl.pallas_call(
        _tr_body,
        grid=(grid,),
        in_specs=[pl.BlockSpec((D, C), lambda g: (0, g))],
        out_specs=pl.BlockSpec((C // 8, 128), lambda g: (g, 0)),
        out_shape=jax.ShapeDtypeStruct((V // 8, 128), jnp.float32),
    )(tT)
    return out.reshape(V, D)


def _remap_idx(i):
    return (i & ~1023) + 8 * (i & 127) + ((i >> 7) & 7)


def _dense_body(pooled_ref, w1_ref, b1_ref, w2_ref, b2_ref, out_ref):
    p = pooled_ref[...]
    h = jnp.maximum(
        jnp.dot(p, w1_ref[...], preferred_element_type=jnp.float32)
        + b1_ref[...], 0.0)
    z = jnp.dot(h, w2_ref[...], preferred_element_type=jnp.float32) + b2_ref[...]
    out_ref[...] = 1.0 / (1.0 + jnp.exp(-z))


@jax.jit
def kernel(indices, table, W1, b1, W2, b2):
    B, L = indices.shape
    D = table.shape[1]
    n_class = W2.shape[1]

    idx2 = _remap_idx(indices.astype(jnp.int32)).reshape(B * L)
    table_rm = _transpose_table(table.T, D)
    sums = _make_sc_pool(B, L, D)(idx2, table_rm)

    out = pl.pallas_call(
        _dense_body,
        out_shape=jax.ShapeDtypeStruct((B, n_class), jnp.float32),
    )(sums, W1 * (1.0 / L), b1.reshape(1, D), W2, b2.reshape(1, n_class))
    return out
